# baseline (speedup 1.0000x reference)
import functools

import jax
import jax.numpy as jnp
import numpy as np
from jax import lax
from jax.experimental import pallas as pl
from jax.experimental.pallas import tpu as pltpu
from jax.experimental.pallas import tpu_sc as plsc

D_MODEL = 64
SEQ_LEN = 200

NUM_CORES = 2
NUM_SUBCORES = 16
NUM_WORKERS = NUM_CORES * NUM_SUBCORES

LANES = 16
CHUNK = 40
POS_PERIOD = SEQ_LEN // CHUNK


def _positional_encoding(depth: int, length: int) -> np.ndarray:
    half = depth / 2
    positions = np.arange(length)[:, np.newaxis]
    depths = np.arange(half)[np.newaxis, :] / half
    angle_rates = 1 / 10000 ** depths
    angle_rads = positions * angle_rates
    return np.concatenate(
        [np.sin(angle_rads), np.cos(angle_rads)], axis=-1
    ).astype(np.float32)


_POS = _positional_encoding(D_MODEL, SEQ_LEN)


def _make_sc_call(total_rows: int):
    rows_per_worker = total_rows // NUM_WORKERS
    n_chunks = rows_per_worker // CHUNK
    mesh = plsc.VectorSubcoreMesh(core_axis_name="c", subcore_axis_name="s")

    @functools.partial(
        pl.kernel,
        mesh=mesh,
        compiler_params=pltpu.CompilerParams(use_tc_tiling_on_sc=False),
        out_type=jax.ShapeDtypeStruct((total_rows, D_MODEL), jnp.float32),
        scratch_types=[
            pltpu.VMEM((rows_per_worker,), jnp.int32),
            pltpu.VMEM((SEQ_LEN, D_MODEL), jnp.float32),
            pltpu.VMEM((CHUNK, D_MODEL), jnp.float32),
        ],
    )
    def sc_call(idx_hbm, table_hbm, pos_hbm, out_hbm, idx_v, pos_v, buf_v):
        wid = lax.axis_index("s") * NUM_CORES + lax.axis_index("c")
        base = wid * rows_per_worker
        pltpu.sync_copy(idx_hbm.at[pl.ds(base, rows_per_worker)], idx_v)
        pltpu.sync_copy(pos_hbm, pos_v)

        @pl.loop(0, n_chunks)
        def _chunk(ch):
            pltpu.sync_copy(
                table_hbm.at[idx_v.at[pl.ds(ch * CHUNK, CHUNK)]], buf_v
            )
            pos_row0 = lax.rem(ch, POS_PERIOD) * CHUNK

            @pl.loop(0, CHUNK)
            def _row(r):
                for j in range(D_MODEL // LANES):
                    cols = pl.ds(j * LANES, LANES)
                    buf_v.at[pl.ds(r, 1), cols][...] = (
                        buf_v.at[pl.ds(r, 1), cols][...] * 8.0
                        + pos_v.at[pl.ds(pos_row0 + r, 1), cols][...]
                    )

            pltpu.sync_copy(
                buf_v, out_hbm.at[pl.ds(base + ch * CHUNK, CHUNK)]
            )

    return sc_call


def kernel(inputs, table):
    n_stack, batch, seq = inputs.shape
    total_rows = n_stack * batch * seq
    idx = inputs.reshape(total_rows).astype(jnp.int32)
    pos = jnp.asarray(_POS)
    out = _make_sc_call(total_rows)(idx, table, pos)
    out = out.reshape(n_stack, batch, seq, D_MODEL)
    return tuple(out[i] for i in range(n_stack))


# speedup vs baseline: 1.1863x; 1.1863x over previous
import functools

import jax
import jax.numpy as jnp
import numpy as np
from jax import lax
from jax.experimental import pallas as pl
from jax.experimental.pallas import tpu as pltpu
from jax.experimental.pallas import tpu_sc as plsc

D_MODEL = 64
SEQ_LEN = 200

NUM_CORES = 2
NUM_SUBCORES = 16
NUM_WORKERS = NUM_CORES * NUM_SUBCORES

LANES = 16
CHUNK = 80
POS_ROWS = 2 * SEQ_LEN
POS_PERIOD = POS_ROWS // CHUNK
NBUF = 4


def _positional_encoding(depth: int, length: int) -> np.ndarray:
    half = depth / 2
    positions = np.arange(length)[:, np.newaxis]
    depths = np.arange(half)[np.newaxis, :] / half
    angle_rates = 1 / 10000 ** depths
    angle_rads = positions * angle_rates
    return np.concatenate(
        [np.sin(angle_rads), np.cos(angle_rads)], axis=-1
    ).astype(np.float32)


_POS = _positional_encoding(D_MODEL, SEQ_LEN)
_POS2 = np.concatenate([_POS, _POS], axis=0)


def _make_sc_call(total_rows: int):
    rows_per_worker = total_rows // NUM_WORKERS
    n_chunks = rows_per_worker // CHUNK
    n_groups = n_chunks // NBUF
    mesh = plsc.VectorSubcoreMesh(core_axis_name="c", subcore_axis_name="s")

    scratch = [
        pltpu.VMEM((rows_per_worker,), jnp.int32),
        pltpu.VMEM((POS_ROWS, D_MODEL), jnp.float32),
    ]
    scratch += [pltpu.VMEM((CHUNK, D_MODEL), jnp.float32)] * NBUF
    scratch += [pltpu.VMEM((CHUNK, D_MODEL), jnp.float32)] * NBUF
    scratch += [pltpu.SemaphoreType.DMA] * (2 * NBUF)

    @functools.partial(
        pl.kernel,
        mesh=mesh,
        compiler_params=pltpu.CompilerParams(use_tc_tiling_on_sc=False),
        out_type=jax.ShapeDtypeStruct((total_rows, D_MODEL), jnp.float32),
        scratch_types=scratch,
    )
    def sc_call(idx_hbm, table_hbm, pos_hbm, out_hbm, idx_v, pos_v, *bufs):
        gbuf = bufs[:NBUF]
        obuf = bufs[NBUF : 2 * NBUF]
        gsem = bufs[2 * NBUF : 3 * NBUF]
        ssem = bufs[3 * NBUF :]

        wid = lax.axis_index("s") * NUM_CORES + lax.axis_index("c")
        base = wid * rows_per_worker
        pltpu.sync_copy(idx_hbm.at[pl.ds(base, rows_per_worker)], idx_v)
        pltpu.sync_copy(pos_hbm, pos_v)

        def start_gather(ch, b):
            pltpu.async_copy(
                table_hbm.at[idx_v.at[pl.ds(ch * CHUNK, CHUNK)]],
                gbuf[b],
                gsem[b],
            )

        for b in range(NBUF):
            start_gather(b, b)

        @pl.loop(0, n_groups)
        def _group(grp):
            for b in range(NBUF):
                ch = grp * NBUF + b
                pltpu.make_async_copy(
                    table_hbm.at[idx_v.at[pl.ds(0, CHUNK)]], gbuf[b], gsem[b]
                ).wait()

                @pl.when(grp > 0)
                def _():
                    pltpu.make_async_copy(
                        obuf[b], out_hbm.at[pl.ds(base, CHUNK)], ssem[b]
                    ).wait()

                pos_row0 = lax.rem(ch, POS_PERIOD) * CHUNK

                @pl.loop(0, CHUNK, step=8)
                def _row(r):
                    for dr in range(8):
                        for j in range(D_MODEL // LANES):
                            rows = pl.ds(r + dr, 1)
                            cols = pl.ds(j * LANES, LANES)
                            obuf[b].at[rows, cols][...] = (
                                gbuf[b].at[rows, cols][...] * 8.0
                                + pos_v.at[pl.ds(pos_row0 + r + dr, 1), cols][...]
                            )

                @pl.when(ch + NBUF < n_chunks)
                def _():
                    start_gather(ch + NBUF, b)

                pltpu.async_copy(
                    obuf[b],
                    out_hbm.at[pl.ds(base + ch * CHUNK, CHUNK)],
                    ssem[b],
                )

        for b in range(NBUF):
            pltpu.make_async_copy(
                obuf[b], out_hbm.at[pl.ds(base, CHUNK)], ssem[b]
            ).wait()

    return sc_call


def kernel(inputs, table):
    n_stack, batch, seq = inputs.shape
    total_rows = n_stack * batch * seq
    idx = inputs.reshape(total_rows).astype(jnp.int32)
    pos = jnp.asarray(_POS2)
    out = _make_sc_call(total_rows)(idx, table, pos)
    out = out.reshape(n_stack, batch, seq, D_MODEL)
    return tuple(out[i] for i in range(n_stack))


# speedup vs baseline: 1.4855x; 1.2522x over previous
import functools

import jax
import jax.numpy as jnp
import numpy as np
from jax import lax
from jax.experimental import pallas as pl
from jax.experimental.pallas import tpu as pltpu
from jax.experimental.pallas import tpu_sc as plsc

D_MODEL = 64
SEQ_LEN = 200

NUM_CORES = 2
NUM_SUBCORES = 16
NUM_WORKERS = NUM_CORES * NUM_SUBCORES

LANES = 16
CHUNK = 80
POS_ROWS = 2 * SEQ_LEN
POS_PERIOD = POS_ROWS // CHUNK
NBUF = 4


def _positional_encoding(depth: int, length: int) -> np.ndarray:
    half = depth / 2
    positions = np.arange(length)[:, np.newaxis]
    depths = np.arange(half)[np.newaxis, :] / half
    angle_rates = 1 / 10000 ** depths
    angle_rads = positions * angle_rates
    return np.concatenate(
        [np.sin(angle_rads), np.cos(angle_rads)], axis=-1
    ).astype(np.float32)


_POS = _positional_encoding(D_MODEL, SEQ_LEN)
_POS2 = np.concatenate([_POS, _POS], axis=0)


def _make_sc_call(total_rows: int):
    half_rows = total_rows // 2
    rows_per_phase = half_rows // NUM_WORKERS
    n_chunks = rows_per_phase // CHUNK
    n_groups = n_chunks // NBUF
    mesh = plsc.VectorSubcoreMesh(core_axis_name="c", subcore_axis_name="s")

    scratch = [
        pltpu.VMEM((rows_per_phase,), jnp.int32),
        pltpu.VMEM((POS_ROWS, D_MODEL), jnp.float32),
    ]
    scratch += [pltpu.VMEM((CHUNK, D_MODEL), jnp.float32)] * NBUF
    scratch += [pltpu.VMEM((CHUNK, D_MODEL), jnp.float32)] * NBUF
    scratch += [pltpu.SemaphoreType.DMA] * (2 * NBUF)

    @functools.partial(
        pl.kernel,
        mesh=mesh,
        compiler_params=pltpu.CompilerParams(use_tc_tiling_on_sc=False),
        out_type=[
            jax.ShapeDtypeStruct((half_rows, D_MODEL), jnp.float32),
            jax.ShapeDtypeStruct((half_rows, D_MODEL), jnp.float32),
        ],
        scratch_types=scratch,
    )
    def sc_call(idx_hbm, table_hbm, pos_hbm, out0_hbm, out1_hbm, idx_v, pos_v, *bufs):
        gbuf = bufs[:NBUF]
        obuf = bufs[NBUF : 2 * NBUF]
        gsem = bufs[2 * NBUF : 3 * NBUF]
        ssem = bufs[3 * NBUF :]

        wid = lax.axis_index("s") * NUM_CORES + lax.axis_index("c")
        lbase = wid * rows_per_phase
        pltpu.sync_copy(pos_hbm, pos_v)

        def run_phase(idx_base, out_hbm):
            pltpu.sync_copy(idx_hbm.at[pl.ds(idx_base, rows_per_phase)], idx_v)

            def start_gather(ch, b):
                pltpu.async_copy(
                    table_hbm.at[idx_v.at[pl.ds(ch * CHUNK, CHUNK)]],
                    gbuf[b],
                    gsem[b],
                )

            for b in range(NBUF):
                start_gather(b, b)

            @pl.loop(0, n_groups)
            def _group(grp):
                for b in range(NBUF):
                    ch = grp * NBUF + b
                    pltpu.make_async_copy(
                        table_hbm.at[idx_v.at[pl.ds(0, CHUNK)]], gbuf[b], gsem[b]
                    ).wait()

                    @pl.when(grp > 0)
                    def _():
                        pltpu.make_async_copy(
                            obuf[b], out_hbm.at[pl.ds(lbase, CHUNK)], ssem[b]
                        ).wait()

                    pos_row0 = lax.rem(ch, POS_PERIOD) * CHUNK

                    @pl.loop(0, CHUNK, step=8)
                    def _row(r):
                        for dr in range(8):
                            for j in range(D_MODEL // LANES):
                                rows = pl.ds(r + dr, 1)
                                cols = pl.ds(j * LANES, LANES)
                                obuf[b].at[rows, cols][...] = (
                                    gbuf[b].at[rows, cols][...] * 8.0
                                    + pos_v.at[
                                        pl.ds(pos_row0 + r + dr, 1), cols
                                    ][...]
                                )

                    @pl.when(ch + NBUF < n_chunks)
                    def _():
                        start_gather(ch + NBUF, b)

                    pltpu.async_copy(
                        obuf[b],
                        out_hbm.at[pl.ds(lbase + ch * CHUNK, CHUNK)],
                        ssem[b],
                    )

            for b in range(NBUF):
                pltpu.make_async_copy(
                    obuf[b], out_hbm.at[pl.ds(lbase, CHUNK)], ssem[b]
                ).wait()

        run_phase(lbase, out0_hbm)
        run_phase(half_rows + lbase, out1_hbm)

    return sc_call


def kernel(inputs, table):
    n_stack, batch, seq = inputs.shape
    total_rows = n_stack * batch * seq
    idx = inputs.reshape(total_rows)
    if idx.dtype != jnp.int32:
        idx = idx.astype(jnp.int32)
    pos = jnp.asarray(_POS2)
    out0, out1 = _make_sc_call(total_rows)(idx, table, pos)
    return (
        out0.reshape(batch, seq, D_MODEL),
        out1.reshape(batch, seq, D_MODEL),
    )
